# Initial kernel scaffold; baseline (speedup 1.0000x reference)
import jax
import jax.numpy as jnp
from jax.experimental import pallas as pl


def kernel(cnn_feature, points, W_mlp0, b_mlp0, W_rnn, b_rnn):
    raise NotImplementedError("write your pallas kernel here")


# speedup vs baseline: 23.9844x; 23.9844x over previous
import functools

import jax
import jax.numpy as jnp
from jax.experimental import pallas as pl

_K = 64
_ITERS = 3
_ROWS = 256


def _adj_kernel(ptsT_ref, ptsR_ref, a_ref):
    ptsT = ptsT_ref[0]
    ptsR = ptsR_ref[0]
    prow0 = ptsT[0:1, :]
    prow1 = ptsT[1:2, :]
    prow2 = ptsT[2:3, :]
    pcol0 = ptsR[:, 0:1]
    pcol1 = ptsR[:, 1:2]
    pcol2 = ptsR[:, 2:3]
    r = pcol0 * prow0 + pcol1 * prow1 + pcol2 * prow2
    diag_row = prow0 * prow0 + prow1 * prow1 + prow2 * prow2
    diag_col = pcol0 * pcol0 + pcol1 * pcol1 + pcol2 * pcol2
    d2 = (diag_col + diag_row) - 2.0 * r
    dist = jnp.sqrt(jnp.maximum(d2, 0.0))
    bits = jax.lax.bitcast_convert_type(dist, jnp.int32)

    rows = bits.shape[0]
    lo0 = jnp.zeros((rows, 1), jnp.int32)
    hi0 = jnp.full((rows, 1), jnp.int32(0x7F7FFFFF), jnp.int32)

    def body(_, carry):
        lo, hi = carry
        mid = lo + (hi - lo) // 2
        cnt = jnp.sum(jnp.where(bits <= mid, 1, 0), axis=1, keepdims=True)
        ge = cnt >= _K
        return jnp.where(ge, lo, mid + 1), jnp.where(ge, mid, hi)

    lo, hi = jax.lax.fori_loop(0, 31, body, (lo0, hi0))
    thresh = hi

    lt = bits < thresh
    eq = bits == thresh
    cnt_lt = jnp.sum(jnp.where(lt, 1, 0), axis=1, keepdims=True)
    cnt_eq = jnp.sum(jnp.where(eq, 1, 0), axis=1, keepdims=True)
    frac = (_K - cnt_lt).astype(jnp.float32) / jnp.maximum(cnt_eq, 1).astype(jnp.float32)
    a_ref[0] = jnp.where(lt, 1.0, jnp.where(eq, frac, 0.0))


def _iter_kernel(adj_ref, h_ref, wmT_ref, bm_ref, wr1T_ref, wr2T_ref, br_ref, out_ref):
    i = pl.program_id(1)
    h = h_ref[0]
    a = jnp.maximum(
        jnp.dot(h, wmT_ref[...], preferred_element_type=jnp.float32,
                precision=jax.lax.Precision.HIGHEST) + bm_ref[...],
        0.0,
    )
    adj = adj_ref[0]
    msg = jnp.dot(adj, a, preferred_element_type=jnp.float32,
                  precision=jax.lax.Precision.HIGHEST) * (1.0 / _K)
    hblk = h_ref[0, pl.ds(i * _ROWS, _ROWS), :]
    out = (
        jnp.dot(hblk, wr1T_ref[...], preferred_element_type=jnp.float32,
                precision=jax.lax.Precision.HIGHEST)
        + jnp.dot(msg, wr2T_ref[...], preferred_element_type=jnp.float32,
                  precision=jax.lax.Precision.HIGHEST)
        + br_ref[...]
    )
    out_ref[0] = jnp.maximum(out, 0.0)


@jax.jit
def kernel(cnn_feature, points, W_mlp0, b_mlp0, W_rnn, b_rnn):
    N, C, H, W = cnn_feature.shape
    HW = H * W
    nblk = HW // _ROWS

    ptsT = points.reshape(N, 3, HW)
    ptsR = ptsT.transpose(0, 2, 1)

    adj = pl.pallas_call(
        _adj_kernel,
        grid=(N, nblk),
        in_specs=[
            pl.BlockSpec((1, 3, HW), lambda n, i: (n, 0, 0)),
            pl.BlockSpec((1, _ROWS, 3), lambda n, i: (n, i, 0)),
        ],
        out_specs=pl.BlockSpec((1, _ROWS, HW), lambda n, i: (n, i, 0)),
        out_shape=jax.ShapeDtypeStruct((N, HW, HW), jnp.float32),
    )(ptsT, ptsR)

    h = cnn_feature.transpose(0, 2, 3, 1).reshape(N, HW, C)
    wmT = W_mlp0.T
    wr1T = W_rnn[:, :C].T
    wr2T = W_rnn[:, C:].T
    bm = b_mlp0.reshape(1, C)
    br = b_rnn.reshape(1, C)

    step = pl.pallas_call(
        _iter_kernel,
        grid=(N, nblk),
        in_specs=[
            pl.BlockSpec((1, _ROWS, HW), lambda n, i: (n, i, 0)),
            pl.BlockSpec((1, HW, C), lambda n, i: (n, 0, 0)),
            pl.BlockSpec((C, C), lambda n, i: (0, 0)),
            pl.BlockSpec((1, C), lambda n, i: (0, 0)),
            pl.BlockSpec((C, C), lambda n, i: (0, 0)),
            pl.BlockSpec((C, C), lambda n, i: (0, 0)),
            pl.BlockSpec((1, C), lambda n, i: (0, 0)),
        ],
        out_specs=pl.BlockSpec((1, _ROWS, C), lambda n, i: (n, i, 0)),
        out_shape=jax.ShapeDtypeStruct((N, HW, C), jnp.float32),
    )
    for _ in range(_ITERS):
        h = step(adj, h, wmT, bm, wr1T, wr2T, br)

    hout = h.reshape(N, H, W, C).transpose(0, 3, 1, 2)
    return jnp.concatenate([cnn_feature, hout], axis=1)


# speedup vs baseline: 35.4984x; 1.4801x over previous
import functools

import jax
import jax.numpy as jnp
from jax.experimental import pallas as pl

_K = 64
_ITERS = 3
_ROWS = 256


def _adj_kernel(ptsT_ref, ptsR_ref, a_ref):
    ptsT = ptsT_ref[0]
    ptsR = ptsR_ref[0]
    prow0 = ptsT[0:1, :]
    prow1 = ptsT[1:2, :]
    prow2 = ptsT[2:3, :]
    pcol0 = ptsR[:, 0:1]
    pcol1 = ptsR[:, 1:2]
    pcol2 = ptsR[:, 2:3]
    r = pcol0 * prow0 + pcol1 * prow1 + pcol2 * prow2
    diag_row = prow0 * prow0 + prow1 * prow1 + prow2 * prow2
    diag_col = pcol0 * pcol0 + pcol1 * pcol1 + pcol2 * pcol2
    d2 = (diag_col + diag_row) - 2.0 * r
    dist = jnp.sqrt(jnp.maximum(d2, 0.0))
    bits = jax.lax.bitcast_convert_type(dist, jnp.int32)

    rows = bits.shape[0]
    lo0 = jnp.zeros((rows, 1), jnp.int32)
    hi0 = jnp.full((rows, 1), jnp.int32(0x7F7FFFFF), jnp.int32)

    ncols = bits.shape[1]

    def body(_, carry):
        lo, hi = carry
        mid = lo + (hi - lo) // 2
        neg = jax.lax.shift_right_arithmetic(bits - (mid + 1), 31)
        parts = [neg[:, c * 128:(c + 1) * 128] for c in range(ncols // 128)]
        while len(parts) > 1:
            parts = [a + b for a, b in zip(parts[::2], parts[1::2])] + (
                [parts[-1]] if len(parts) % 2 else [])
        cnt = -jnp.sum(parts[0], axis=1, keepdims=True)
        ge = cnt >= _K
        return jnp.where(ge, lo, mid + 1), jnp.where(ge, mid, hi)

    lo, hi = jax.lax.fori_loop(0, 31, body, (lo0, hi0))
    thresh = hi

    lt = bits < thresh
    eq = bits == thresh
    cnt_lt = jnp.sum(jnp.where(lt, 1, 0), axis=1, keepdims=True)
    cnt_eq = jnp.sum(jnp.where(eq, 1, 0), axis=1, keepdims=True)
    frac = (_K - cnt_lt).astype(jnp.float32) / jnp.maximum(cnt_eq, 1).astype(jnp.float32)
    a_ref[0] = jnp.where(lt, 1.0, jnp.where(eq, frac, 0.0))


def _iter_kernel(adj_ref, h_ref, wmT_ref, bm_ref, wr1T_ref, wr2T_ref, br_ref, out_ref):
    i = pl.program_id(1)
    h = h_ref[0]
    a = jnp.maximum(
        jnp.dot(h, wmT_ref[...], preferred_element_type=jnp.float32,
                precision=jax.lax.Precision.DEFAULT) + bm_ref[...],
        0.0,
    )
    adj = adj_ref[0]
    msg = jnp.dot(adj, a, preferred_element_type=jnp.float32,
                  precision=jax.lax.Precision.DEFAULT) * (1.0 / _K)
    hblk = h_ref[0, pl.ds(i * _ROWS, _ROWS), :]
    out = (
        jnp.dot(hblk, wr1T_ref[...], preferred_element_type=jnp.float32,
                precision=jax.lax.Precision.DEFAULT)
        + jnp.dot(msg, wr2T_ref[...], preferred_element_type=jnp.float32,
                  precision=jax.lax.Precision.DEFAULT)
        + br_ref[...]
    )
    out_ref[0] = jnp.maximum(out, 0.0)


@jax.jit
def kernel(cnn_feature, points, W_mlp0, b_mlp0, W_rnn, b_rnn):
    N, C, H, W = cnn_feature.shape
    HW = H * W
    nblk = HW // _ROWS

    ptsT = points.reshape(N, 3, HW)
    ptsR = ptsT.transpose(0, 2, 1)

    adj = pl.pallas_call(
        _adj_kernel,
        grid=(N, nblk),
        in_specs=[
            pl.BlockSpec((1, 3, HW), lambda n, i: (n, 0, 0)),
            pl.BlockSpec((1, _ROWS, 3), lambda n, i: (n, i, 0)),
        ],
        out_specs=pl.BlockSpec((1, _ROWS, HW), lambda n, i: (n, i, 0)),
        out_shape=jax.ShapeDtypeStruct((N, HW, HW), jnp.float32),
    )(ptsT, ptsR)

    h = cnn_feature.transpose(0, 2, 3, 1).reshape(N, HW, C)
    wmT = W_mlp0.T
    wr1T = W_rnn[:, :C].T
    wr2T = W_rnn[:, C:].T
    bm = b_mlp0.reshape(1, C)
    br = b_rnn.reshape(1, C)

    step = pl.pallas_call(
        _iter_kernel,
        grid=(N, nblk),
        in_specs=[
            pl.BlockSpec((1, _ROWS, HW), lambda n, i: (n, i, 0)),
            pl.BlockSpec((1, HW, C), lambda n, i: (n, 0, 0)),
            pl.BlockSpec((C, C), lambda n, i: (0, 0)),
            pl.BlockSpec((1, C), lambda n, i: (0, 0)),
            pl.BlockSpec((C, C), lambda n, i: (0, 0)),
            pl.BlockSpec((C, C), lambda n, i: (0, 0)),
            pl.BlockSpec((1, C), lambda n, i: (0, 0)),
        ],
        out_specs=pl.BlockSpec((1, _ROWS, C), lambda n, i: (n, i, 0)),
        out_shape=jax.ShapeDtypeStruct((N, HW, C), jnp.float32),
    )
    for _ in range(_ITERS):
        h = step(adj, h, wmT, bm, wr1T, wr2T, br)

    hout = h.reshape(N, H, W, C).transpose(0, 3, 1, 2)
    return jnp.concatenate([cnn_feature, hout], axis=1)


# speedup vs baseline: 39.6198x; 1.1161x over previous
import jax
import jax.numpy as jnp
from jax.experimental import pallas as pl
from jax.experimental.pallas import tpu as pltpu

_K = 64
_ITERS = 3
_ROWS = 256


def _adj_block(ptsT, ptsR_ref, b):
    rb = pl.multiple_of(b * _ROWS, _ROWS)
    prow0 = ptsT[0:1, :]
    prow1 = ptsT[1:2, :]
    prow2 = ptsT[2:3, :]
    pcol0 = ptsR_ref[0, pl.ds(rb, _ROWS), 0:1]
    pcol1 = ptsR_ref[0, pl.ds(rb, _ROWS), 1:2]
    pcol2 = ptsR_ref[0, pl.ds(rb, _ROWS), 2:3]
    r = pcol0 * prow0 + pcol1 * prow1 + pcol2 * prow2
    diag_row = prow0 * prow0 + prow1 * prow1 + prow2 * prow2
    diag_col = pcol0 * pcol0 + pcol1 * pcol1 + pcol2 * pcol2
    d2 = (diag_col + diag_row) - 2.0 * r
    dist = jnp.sqrt(jnp.maximum(d2, 0.0))
    bits = jax.lax.bitcast_convert_type(dist, jnp.int32)

    lo0 = jnp.zeros((_ROWS, 1), jnp.int32)
    hi0 = jnp.full((_ROWS, 1), jnp.int32(0x7F7FFFFF), jnp.int32)
    nchunk = bits.shape[1] // 128

    def body(_, carry):
        lo, hi = carry
        mid = lo + (hi - lo) // 2
        neg = jax.lax.shift_right_arithmetic(bits - (mid + 1), 31)
        parts = [neg[:, c * 128:(c + 1) * 128] for c in range(nchunk)]
        while len(parts) > 1:
            parts = [x + y for x, y in zip(parts[::2], parts[1::2])] + (
                [parts[-1]] if len(parts) % 2 else [])
        cnt = -jnp.sum(parts[0], axis=1, keepdims=True)
        ge = cnt >= _K
        return jnp.where(ge, lo, mid + 1), jnp.where(ge, mid, hi)

    _, thresh = jax.lax.fori_loop(0, 31, body, (lo0, hi0))

    lt = bits < thresh
    eq = bits == thresh
    cnt_lt = jnp.sum(jnp.where(lt, 1, 0), axis=1, keepdims=True)
    cnt_eq = jnp.sum(jnp.where(eq, 1, 0), axis=1, keepdims=True)
    frac = (_K - cnt_lt).astype(jnp.float32) / jnp.maximum(cnt_eq, 1).astype(jnp.float32)
    return jnp.where(lt, 1.0, jnp.where(eq, frac, 0.0))


def _fused_kernel(ptsT_ref, ptsR_ref, h0_ref, wmT_ref, bm_ref, wr1T_ref,
                  wr2T_ref, br_ref, out_ref, adj_ref, g_ref, h1_ref, h2_ref):
    HW = ptsT_ref.shape[2]
    nblk = HW // _ROWS
    ptsT = ptsT_ref[0]

    def adj_body(b, _):
        rb = pl.multiple_of(b * _ROWS, _ROWS)
        adj_ref[pl.ds(rb, _ROWS), :] = _adj_block(ptsT, ptsR_ref, b)
        return 0

    jax.lax.fori_loop(0, nblk, adj_body, 0)

    srcs = [lambda sl: h0_ref[0, sl, :], lambda sl: h1_ref[sl, :],
            lambda sl: h2_ref[sl, :]]
    dsts = [lambda sl, v: h1_ref.__setitem__((sl, slice(None)), v),
            lambda sl, v: h2_ref.__setitem__((sl, slice(None)), v),
            lambda sl, v: out_ref.__setitem__((0, sl, slice(None)), v)]
    full = pl.ds(0, HW)
    for t in range(_ITERS):
        src, dst = srcs[t], dsts[t]
        g_ref[...] = jnp.maximum(
            jnp.dot(src(full), wmT_ref[...],
                    preferred_element_type=jnp.float32) + bm_ref[...],
            0.0,
        )
        g = g_ref[...]

        def msg_body(b, _):
            rb = pl.multiple_of(b * _ROWS, _ROWS)
            sl = pl.ds(rb, _ROWS)
            msg = jnp.dot(adj_ref[sl, :], g,
                          preferred_element_type=jnp.float32) * (1.0 / _K)
            out = (
                jnp.dot(src(sl), wr1T_ref[...],
                        preferred_element_type=jnp.float32)
                + jnp.dot(msg, wr2T_ref[...],
                          preferred_element_type=jnp.float32)
                + br_ref[...]
            )
            dst(sl, jnp.maximum(out, 0.0))
            return 0

        jax.lax.fori_loop(0, nblk, msg_body, 0)


@jax.jit
def kernel(cnn_feature, points, W_mlp0, b_mlp0, W_rnn, b_rnn):
    N, C, H, W = cnn_feature.shape
    HW = H * W

    ptsT = points.reshape(N, 3, HW)
    ptsR = ptsT.transpose(0, 2, 1)
    h0 = cnn_feature.transpose(0, 2, 3, 1).reshape(N, HW, C)
    wmT = W_mlp0.T
    wr1T = W_rnn[:, :C].T
    wr2T = W_rnn[:, C:].T
    bm = b_mlp0.reshape(1, C)
    br = b_rnn.reshape(1, C)

    h = pl.pallas_call(
        _fused_kernel,
        grid=(N,),
        in_specs=[
            pl.BlockSpec((1, 3, HW), lambda n: (n, 0, 0)),
            pl.BlockSpec((1, HW, 3), lambda n: (n, 0, 0)),
            pl.BlockSpec((1, HW, C), lambda n: (n, 0, 0)),
            pl.BlockSpec((C, C), lambda n: (0, 0)),
            pl.BlockSpec((1, C), lambda n: (0, 0)),
            pl.BlockSpec((C, C), lambda n: (0, 0)),
            pl.BlockSpec((C, C), lambda n: (0, 0)),
            pl.BlockSpec((1, C), lambda n: (0, 0)),
        ],
        out_specs=pl.BlockSpec((1, HW, C), lambda n: (n, 0, 0)),
        out_shape=jax.ShapeDtypeStruct((N, HW, C), jnp.float32),
        scratch_shapes=[
            pltpu.VMEM((HW, HW), jnp.float32),
            pltpu.VMEM((HW, C), jnp.float32),
            pltpu.VMEM((HW, C), jnp.float32),
            pltpu.VMEM((HW, C), jnp.float32),
        ],
    )(ptsT, ptsR, h0, wmT, bm, wr1T, wr2T, br)

    hout = h.reshape(N, H, W, C).transpose(0, 3, 1, 2)
    return jnp.concatenate([cnn_feature, hout], axis=1)


# speedup vs baseline: 47.9846x; 1.2111x over previous
import jax
import jax.numpy as jnp
from jax.experimental import pallas as pl
from jax.experimental.pallas import tpu as pltpu

_K = 64
_ITERS = 3
_ROWS = 256


def _adj_block(ptsT, ptsR_ref, b):
    rb = pl.multiple_of(b * _ROWS, _ROWS)
    prow0 = ptsT[0:1, :]
    prow1 = ptsT[1:2, :]
    prow2 = ptsT[2:3, :]
    pcol0 = ptsR_ref[0, pl.ds(rb, _ROWS), 0:1]
    pcol1 = ptsR_ref[0, pl.ds(rb, _ROWS), 1:2]
    pcol2 = ptsR_ref[0, pl.ds(rb, _ROWS), 2:3]
    r = pcol0 * prow0 + pcol1 * prow1 + pcol2 * prow2
    diag_row = prow0 * prow0 + prow1 * prow1 + prow2 * prow2
    diag_col = pcol0 * pcol0 + pcol1 * pcol1 + pcol2 * pcol2
    d2 = (diag_col + diag_row) - 2.0 * r
    dist = jnp.sqrt(jnp.maximum(d2, 0.0))
    bits = jax.lax.bitcast_convert_type(dist, jnp.int32)

    def i16_count(vals16, m):
        neg = jnp.where(vals16 <= m, jnp.int16(-1), jnp.int16(0))
        parts = [neg[:, c * 256:(c + 1) * 256]
                 for c in range(vals16.shape[1] // 256)]
        while len(parts) > 1:
            parts = [x + y for x, y in zip(parts[::2], parts[1::2])] + (
                [parts[-1]] if len(parts) % 2 else [])
        return -jnp.sum(parts[0].astype(jnp.int32), axis=1, keepdims=True)

    def count_search(vals16, iters, hi_init, base):
        lo0 = jnp.zeros((_ROWS, 1), jnp.int32)
        hi0 = jnp.full((_ROWS, 1), jnp.int32(hi_init), jnp.int32)

        def body(_, carry):
            lo, hi = carry
            mid = lo + ((hi - lo) >> 1)
            ge = (base + i16_count(vals16, mid.astype(jnp.int16))) >= _K
            return jnp.where(ge, lo, mid + 1), jnp.where(ge, mid, hi)

        _, t = jax.lax.fori_loop(0, iters, body, (lo0, hi0))
        return t

    hi16 = jax.lax.shift_right_logical(bits, 16).astype(jnp.int16)
    t_hi = count_search(hi16, 16, 0x7F7F, 0)

    cnt_base = i16_count(hi16, (t_hi - 1).astype(jnp.int16))
    match = hi16 == t_hi.astype(jnp.int16)
    sentinel = jnp.int16(32767)

    b15 = jax.lax.shift_right_logical(bits, 15).astype(jnp.int16) & jnp.int16(1)
    vals_b = jnp.where(match, b15, sentinel)
    cnt0 = cnt_base + i16_count(vals_b, jnp.int16(0))
    ge0 = cnt0 >= _K
    t_b15 = jnp.where(ge0, 0, 1)
    cnt_base2 = jnp.where(ge0, cnt_base, cnt0)

    low15 = (bits & 0x7FFF).astype(jnp.int16)
    vals_l = jnp.where(match & (b15 == t_b15.astype(jnp.int16)), low15, sentinel)
    t_low = count_search(vals_l, 15, 0x7FFF, cnt_base2)

    thresh = (
        jax.lax.shift_left(t_hi, 16)
        | jax.lax.shift_left(t_b15, 15)
        | t_low
    )

    lt = bits < thresh
    eq = bits == thresh
    cnt_lt = jnp.sum(jnp.where(lt, 1, 0), axis=1, keepdims=True)
    cnt_eq = jnp.sum(jnp.where(eq, 1, 0), axis=1, keepdims=True)
    frac = (_K - cnt_lt).astype(jnp.float32) / jnp.maximum(cnt_eq, 1).astype(jnp.float32)
    return jnp.where(lt, 1.0, jnp.where(eq, frac, 0.0))


def _fused_kernel(ptsT_ref, ptsR_ref, h0_ref, wmT_ref, bm_ref, wr1T_ref,
                  wr2T_ref, br_ref, out_ref, adj_ref, g_ref, h1_ref, h2_ref):
    HW = ptsT_ref.shape[2]
    nblk = HW // _ROWS
    ptsT = ptsT_ref[0]

    def adj_body(b, _):
        rb = pl.multiple_of(b * _ROWS, _ROWS)
        adj_ref[pl.ds(rb, _ROWS), :] = _adj_block(ptsT, ptsR_ref, b)
        return 0

    jax.lax.fori_loop(0, nblk, adj_body, 0)

    srcs = [lambda sl: h0_ref[0, sl, :], lambda sl: h1_ref[sl, :],
            lambda sl: h2_ref[sl, :]]
    dsts = [lambda sl, v: h1_ref.__setitem__((sl, slice(None)), v),
            lambda sl, v: h2_ref.__setitem__((sl, slice(None)), v),
            lambda sl, v: out_ref.__setitem__((0, sl, slice(None)), v)]
    full = pl.ds(0, HW)
    for t in range(_ITERS):
        src, dst = srcs[t], dsts[t]
        g_ref[...] = jnp.maximum(
            jnp.dot(src(full), wmT_ref[...],
                    preferred_element_type=jnp.float32) + bm_ref[...],
            0.0,
        )
        g = g_ref[...]

        def msg_body(b, _):
            rb = pl.multiple_of(b * _ROWS, _ROWS)
            sl = pl.ds(rb, _ROWS)
            msg = jnp.dot(adj_ref[sl, :], g,
                          preferred_element_type=jnp.float32) * (1.0 / _K)
            out = (
                jnp.dot(src(sl), wr1T_ref[...],
                        preferred_element_type=jnp.float32)
                + jnp.dot(msg, wr2T_ref[...],
                          preferred_element_type=jnp.float32)
                + br_ref[...]
            )
            dst(sl, jnp.maximum(out, 0.0))
            return 0

        jax.lax.fori_loop(0, nblk, msg_body, 0)


@jax.jit
def kernel(cnn_feature, points, W_mlp0, b_mlp0, W_rnn, b_rnn):
    N, C, H, W = cnn_feature.shape
    HW = H * W

    ptsT = points.reshape(N, 3, HW)
    ptsR = ptsT.transpose(0, 2, 1)
    h0 = cnn_feature.transpose(0, 2, 3, 1).reshape(N, HW, C)
    wmT = W_mlp0.T
    wr1T = W_rnn[:, :C].T
    wr2T = W_rnn[:, C:].T
    bm = b_mlp0.reshape(1, C)
    br = b_rnn.reshape(1, C)

    h = pl.pallas_call(
        _fused_kernel,
        grid=(N,),
        in_specs=[
            pl.BlockSpec((1, 3, HW), lambda n: (n, 0, 0)),
            pl.BlockSpec((1, HW, 3), lambda n: (n, 0, 0)),
            pl.BlockSpec((1, HW, C), lambda n: (n, 0, 0)),
            pl.BlockSpec((C, C), lambda n: (0, 0)),
            pl.BlockSpec((1, C), lambda n: (0, 0)),
            pl.BlockSpec((C, C), lambda n: (0, 0)),
            pl.BlockSpec((C, C), lambda n: (0, 0)),
            pl.BlockSpec((1, C), lambda n: (0, 0)),
        ],
        out_specs=pl.BlockSpec((1, HW, C), lambda n: (n, 0, 0)),
        out_shape=jax.ShapeDtypeStruct((N, HW, C), jnp.float32),
        scratch_shapes=[
            pltpu.VMEM((HW, HW), jnp.float32),
            pltpu.VMEM((HW, C), jnp.float32),
            pltpu.VMEM((HW, C), jnp.float32),
            pltpu.VMEM((HW, C), jnp.float32),
        ],
    )(ptsT, ptsR, h0, wmT, bm, wr1T, wr2T, br)

    hout = h.reshape(N, H, W, C).transpose(0, 3, 1, 2)
    return jnp.concatenate([cnn_feature, hout], axis=1)


# speedup vs baseline: 60.8897x; 1.2689x over previous
import jax
import jax.numpy as jnp
from jax.experimental import pallas as pl
from jax.experimental.pallas import tpu as pltpu

_K = 64
_ITERS = 3
_ROWS = 256


def _adj_block(ptsT, ptsR_ref, b):
    rb = pl.multiple_of(b * _ROWS, _ROWS)
    prow0 = ptsT[0:1, :]
    prow1 = ptsT[1:2, :]
    prow2 = ptsT[2:3, :]
    pcol0 = ptsR_ref[0, pl.ds(rb, _ROWS), 0:1]
    pcol1 = ptsR_ref[0, pl.ds(rb, _ROWS), 1:2]
    pcol2 = ptsR_ref[0, pl.ds(rb, _ROWS), 2:3]
    r = pcol0 * prow0 + pcol1 * prow1 + pcol2 * prow2
    diag_row = prow0 * prow0 + prow1 * prow1 + prow2 * prow2
    diag_col = pcol0 * pcol0 + pcol1 * pcol1 + pcol2 * pcol2
    d2 = (diag_col + diag_row) - 2.0 * r
    dist = jnp.sqrt(jnp.maximum(d2, 0.0))
    bits = jax.lax.bitcast_convert_type(dist, jnp.int32)

    def i16_count(vals16, m):
        neg = jnp.where(vals16 <= m, jnp.int16(-1), jnp.int16(0))
        parts = [neg[:, c * 256:(c + 1) * 256]
                 for c in range(vals16.shape[1] // 256)]
        while len(parts) > 1:
            parts = [x + y for x, y in zip(parts[::2], parts[1::2])] + (
                [parts[-1]] if len(parts) % 2 else [])
        return -jnp.sum(parts[0].astype(jnp.int32), axis=1, keepdims=True)

    def count_search(vals16, iters, hi_init, base):
        lo0 = jnp.zeros((_ROWS, 1), jnp.int32)
        hi0 = jnp.full((_ROWS, 1), jnp.int32(hi_init), jnp.int32)

        def body(_, carry):
            lo, hi = carry
            mid = lo + ((hi - lo) >> 1)
            ge = (base + i16_count(vals16, mid.astype(jnp.int16))) >= _K
            return jnp.where(ge, lo, mid + 1), jnp.where(ge, mid, hi)

        _, t = jax.lax.fori_loop(0, iters, body, (lo0, hi0))
        return t

    hi16 = jax.lax.shift_right_logical(bits, 16).astype(jnp.int16)
    t_hi = count_search(hi16, 15, 0x7F7F, 0)

    cnt_base = i16_count(hi16, (t_hi - 1).astype(jnp.int16))
    match = hi16 == t_hi.astype(jnp.int16)
    sentinel = jnp.int16(32767)

    b15 = jax.lax.shift_right_logical(bits, 15).astype(jnp.int16) & jnp.int16(1)
    vals_b = jnp.where(match, b15, sentinel)
    cnt0 = cnt_base + i16_count(vals_b, jnp.int16(0))
    ge0 = cnt0 >= _K
    t_b15 = jnp.where(ge0, 0, 1)
    cnt_base2 = jnp.where(ge0, cnt_base, cnt0)

    low15 = (bits & 0x7FFF).astype(jnp.int16)
    vals_l = jnp.where(match & (b15 == t_b15.astype(jnp.int16)), low15, sentinel)
    t_low = count_search(vals_l, 9, 0x7FFF, cnt_base2)

    thresh = (
        jax.lax.shift_left(t_hi, 16)
        | jax.lax.shift_left(t_b15, 15)
        | t_low
    )

    cnt_lt = cnt_base2 + i16_count(vals_l, (t_low - 1).astype(jnp.int16))
    cnt_le = cnt_base2 + i16_count(vals_l, t_low.astype(jnp.int16))
    cnt_eq = cnt_le - cnt_lt
    frac = (_K - cnt_lt).astype(jnp.float32) / jnp.maximum(cnt_eq, 1).astype(jnp.float32)
    lt = bits < thresh
    eq = bits == thresh
    return jnp.where(lt, 1.0, jnp.where(eq, frac, 0.0))


def _fused_kernel(ptsT_ref, ptsR_ref, h0_ref, wmT_ref, bm_ref, wr1T_ref,
                  wr2T_ref, br_ref, out_ref, adj_ref, g_ref, h1_ref, h2_ref):
    HW = ptsT_ref.shape[2]
    nblk = HW // _ROWS
    ptsT = ptsT_ref[0]

    def adj_body(b, _):
        rb = pl.multiple_of(b * _ROWS, _ROWS)
        adj_ref[pl.ds(rb, _ROWS), :] = _adj_block(ptsT, ptsR_ref, b)
        return 0

    jax.lax.fori_loop(0, nblk, adj_body, 0)

    srcs = [lambda sl: h0_ref[0, sl, :], lambda sl: h1_ref[sl, :],
            lambda sl: h2_ref[sl, :]]
    dsts = [lambda sl, v: h1_ref.__setitem__((sl, slice(None)), v),
            lambda sl, v: h2_ref.__setitem__((sl, slice(None)), v),
            lambda sl, v: out_ref.__setitem__((0, sl, slice(None)), v)]
    full = pl.ds(0, HW)
    for t in range(_ITERS):
        src, dst = srcs[t], dsts[t]
        g_ref[...] = jnp.maximum(
            jnp.dot(src(full), wmT_ref[...],
                    preferred_element_type=jnp.float32) + bm_ref[...],
            0.0,
        )
        g = g_ref[...]

        def msg_body(b, _):
            rb = pl.multiple_of(b * _ROWS, _ROWS)
            sl = pl.ds(rb, _ROWS)
            msg = jnp.dot(adj_ref[sl, :], g,
                          preferred_element_type=jnp.float32) * (1.0 / _K)
            out = (
                jnp.dot(src(sl), wr1T_ref[...],
                        preferred_element_type=jnp.float32)
                + jnp.dot(msg, wr2T_ref[...],
                          preferred_element_type=jnp.float32)
                + br_ref[...]
            )
            dst(sl, jnp.maximum(out, 0.0))
            return 0

        jax.lax.fori_loop(0, nblk, msg_body, 0)


@jax.jit
def kernel(cnn_feature, points, W_mlp0, b_mlp0, W_rnn, b_rnn):
    N, C, H, W = cnn_feature.shape
    HW = H * W

    ptsT = points.reshape(N, 3, HW)
    ptsR = ptsT.transpose(0, 2, 1)
    h0 = cnn_feature.transpose(0, 2, 3, 1).reshape(N, HW, C)
    wmT = W_mlp0.T
    wr1T = W_rnn[:, :C].T
    wr2T = W_rnn[:, C:].T
    bm = b_mlp0.reshape(1, C)
    br = b_rnn.reshape(1, C)

    h = pl.pallas_call(
        _fused_kernel,
        grid=(N,),
        in_specs=[
            pl.BlockSpec((1, 3, HW), lambda n: (n, 0, 0)),
            pl.BlockSpec((1, HW, 3), lambda n: (n, 0, 0)),
            pl.BlockSpec((1, HW, C), lambda n: (n, 0, 0)),
            pl.BlockSpec((C, C), lambda n: (0, 0)),
            pl.BlockSpec((1, C), lambda n: (0, 0)),
            pl.BlockSpec((C, C), lambda n: (0, 0)),
            pl.BlockSpec((C, C), lambda n: (0, 0)),
            pl.BlockSpec((1, C), lambda n: (0, 0)),
        ],
        out_specs=pl.BlockSpec((1, HW, C), lambda n: (n, 0, 0)),
        out_shape=jax.ShapeDtypeStruct((N, HW, C), jnp.float32),
        scratch_shapes=[
            pltpu.VMEM((HW, HW), jnp.float32),
            pltpu.VMEM((HW, C), jnp.float32),
            pltpu.VMEM((HW, C), jnp.float32),
            pltpu.VMEM((HW, C), jnp.float32),
        ],
    )(ptsT, ptsR, h0, wmT, bm, wr1T, wr2T, br)

    hout = h.reshape(N, H, W, C).transpose(0, 3, 1, 2)
    return jnp.concatenate([cnn_feature, hout], axis=1)


# speedup vs baseline: 65.2745x; 1.0720x over previous
import jax
import jax.numpy as jnp
from jax.experimental import pallas as pl
from jax.experimental.pallas import tpu as pltpu

_K = 64
_ITERS = 3
_ROWS = 256


def _adj_block(ptsT, ptsR_ref, b):
    rb = pl.multiple_of(b * _ROWS, _ROWS)
    prow0 = ptsT[0:1, :]
    prow1 = ptsT[1:2, :]
    prow2 = ptsT[2:3, :]
    pblk = ptsR_ref[0, pl.ds(rb, _ROWS), :]
    pcol0 = pblk[:, 0:1]
    pcol1 = pblk[:, 1:2]
    pcol2 = pblk[:, 2:3]
    r = jnp.dot(pblk, ptsT, preferred_element_type=jnp.float32)
    diag_row = prow0 * prow0 + prow1 * prow1 + prow2 * prow2
    diag_col = pcol0 * pcol0 + pcol1 * pcol1 + pcol2 * pcol2
    d2 = jnp.maximum((diag_col + diag_row) - 2.0 * r, 0.0)
    bits = jax.lax.bitcast_convert_type(d2, jnp.int32)

    def i16_count(vals16, m):
        neg = jnp.where(vals16 <= m, jnp.int16(-1), jnp.int16(0))
        parts = [neg[:, c * 256:(c + 1) * 256]
                 for c in range(vals16.shape[1] // 256)]
        while len(parts) > 1:
            parts = [x + y for x, y in zip(parts[::2], parts[1::2])] + (
                [parts[-1]] if len(parts) % 2 else [])
        return -jnp.sum(parts[0], axis=1, keepdims=True).astype(jnp.int32)

    def count_search(vals16, iters, hi_init, base):
        lo0 = jnp.zeros((_ROWS, 1), jnp.int32)
        hi0 = jnp.full((_ROWS, 1), jnp.int32(hi_init), jnp.int32)

        def body(_, carry):
            lo, hi = carry
            mid = lo + ((hi - lo) >> 1)
            ge = (base + i16_count(vals16, mid.astype(jnp.int16))) >= _K
            return jnp.where(ge, lo, mid + 1), jnp.where(ge, mid, hi)

        _, t = jax.lax.fori_loop(0, iters, body, (lo0, hi0))
        return t

    hi16 = jax.lax.shift_right_logical(bits, 16).astype(jnp.int16)
    t_hi = count_search(hi16, 15, 0x7F7F, 0)

    cnt_base = i16_count(hi16, (t_hi - 1).astype(jnp.int16))
    match = hi16 == t_hi.astype(jnp.int16)
    sentinel = jnp.int16(32767)

    b15 = jax.lax.shift_right_logical(bits, 15).astype(jnp.int16) & jnp.int16(1)
    vals_b = jnp.where(match, b15, sentinel)
    cnt0 = cnt_base + i16_count(vals_b, jnp.int16(0))
    ge0 = cnt0 >= _K
    t_b15 = jnp.where(ge0, 0, 1)
    cnt_base2 = jnp.where(ge0, cnt_base, cnt0)

    low15 = (bits & 0x7FFF).astype(jnp.int16)
    vals_l = jnp.where(match & (b15 == t_b15.astype(jnp.int16)), low15, sentinel)
    t_low = count_search(vals_l, 9, 0x7FFF, cnt_base2)

    thresh = (
        jax.lax.shift_left(t_hi, 16)
        | jax.lax.shift_left(t_b15, 15)
        | t_low
    )

    cnt_lt = cnt_base2 + i16_count(vals_l, (t_low - 1).astype(jnp.int16))
    cnt_le = cnt_base2 + i16_count(vals_l, t_low.astype(jnp.int16))
    cnt_eq = cnt_le - cnt_lt
    frac = (_K - cnt_lt).astype(jnp.float32) / jnp.maximum(cnt_eq, 1).astype(jnp.float32)
    lt = bits < thresh
    eq = bits == thresh
    return jnp.where(lt, 1.0, jnp.where(eq, frac, 0.0))


def _fused_kernel(ptsT_ref, ptsR_ref, h0_ref, wmT_ref, bm_ref, wr1T_ref,
                  wr2T_ref, br_ref, out_ref, adj_ref, g_ref, h1_ref, h2_ref):
    HW = ptsT_ref.shape[2]
    nblk = HW // _ROWS
    ptsT = ptsT_ref[0]

    def adj_body(b, _):
        rb = pl.multiple_of(b * _ROWS, _ROWS)
        adj_ref[pl.ds(rb, _ROWS), :] = _adj_block(ptsT, ptsR_ref, b)
        return 0

    jax.lax.fori_loop(0, nblk, adj_body, 0)

    srcs = [lambda sl: h0_ref[0, sl, :], lambda sl: h1_ref[sl, :],
            lambda sl: h2_ref[sl, :]]
    dsts = [lambda sl, v: h1_ref.__setitem__((sl, slice(None)), v),
            lambda sl, v: h2_ref.__setitem__((sl, slice(None)), v),
            lambda sl, v: out_ref.__setitem__((0, sl, slice(None)), v)]
    full = pl.ds(0, HW)
    for t in range(_ITERS):
        src, dst = srcs[t], dsts[t]
        g_ref[...] = jnp.maximum(
            jnp.dot(src(full), wmT_ref[...],
                    preferred_element_type=jnp.float32) + bm_ref[...],
            0.0,
        )
        g = g_ref[...]

        def msg_body(b, _):
            rb = pl.multiple_of(b * _ROWS, _ROWS)
            sl = pl.ds(rb, _ROWS)
            msg = jnp.dot(adj_ref[sl, :], g,
                          preferred_element_type=jnp.float32) * (1.0 / _K)
            out = (
                jnp.dot(src(sl), wr1T_ref[...],
                        preferred_element_type=jnp.float32)
                + jnp.dot(msg, wr2T_ref[...],
                          preferred_element_type=jnp.float32)
                + br_ref[...]
            )
            dst(sl, jnp.maximum(out, 0.0))
            return 0

        jax.lax.fori_loop(0, nblk, msg_body, 0)


@jax.jit
def kernel(cnn_feature, points, W_mlp0, b_mlp0, W_rnn, b_rnn):
    N, C, H, W = cnn_feature.shape
    HW = H * W

    ptsT = points.reshape(N, 3, HW)
    ptsR = ptsT.transpose(0, 2, 1)
    h0 = cnn_feature.transpose(0, 2, 3, 1).reshape(N, HW, C)
    wmT = W_mlp0.T
    wr1T = W_rnn[:, :C].T
    wr2T = W_rnn[:, C:].T
    bm = b_mlp0.reshape(1, C)
    br = b_rnn.reshape(1, C)

    h = pl.pallas_call(
        _fused_kernel,
        grid=(N,),
        in_specs=[
            pl.BlockSpec((1, 3, HW), lambda n: (n, 0, 0)),
            pl.BlockSpec((1, HW, 3), lambda n: (n, 0, 0)),
            pl.BlockSpec((1, HW, C), lambda n: (n, 0, 0)),
            pl.BlockSpec((C, C), lambda n: (0, 0)),
            pl.BlockSpec((1, C), lambda n: (0, 0)),
            pl.BlockSpec((C, C), lambda n: (0, 0)),
            pl.BlockSpec((C, C), lambda n: (0, 0)),
            pl.BlockSpec((1, C), lambda n: (0, 0)),
        ],
        out_specs=pl.BlockSpec((1, HW, C), lambda n: (n, 0, 0)),
        out_shape=jax.ShapeDtypeStruct((N, HW, C), jnp.float32),
        scratch_shapes=[
            pltpu.VMEM((HW, HW), jnp.float32),
            pltpu.VMEM((HW, C), jnp.float32),
            pltpu.VMEM((HW, C), jnp.float32),
            pltpu.VMEM((HW, C), jnp.float32),
        ],
    )(ptsT, ptsR, h0, wmT, bm, wr1T, wr2T, br)

    hout = h.reshape(N, H, W, C).transpose(0, 3, 1, 2)
    return jnp.concatenate([cnn_feature, hout], axis=1)
